# Initial kernel scaffold; baseline (speedup 1.0000x reference)
import jax
import jax.numpy as jnp
from jax.experimental import pallas as pl


def kernel(x, edge_index, edge_attr, fc1_w, fc1_b, k1_w, k1_b, k2_w, k2_b, k3_w, k3_b, fc2_w, fc2_b, fc3_w, fc3_b):
    raise NotImplementedError("write your pallas kernel here")


# speedup vs baseline: 3.0409x; 3.0409x over previous
import functools

import jax
import jax.numpy as jnp
from jax import lax
from jax.experimental import pallas as pl
from jax.experimental.pallas import tpu as pltpu
from jax.experimental.pallas import tpu_sc as plsc

N = 10000
E = 320000
WIDTH = 16
KER_WIDTH = 128
DEPTH = 4

NW = 32
CHUNK = 128
CPW = 80
EPW = CHUNK * CPW
EP = NW * EPW
NP = 10240
ROWS_PER_TILE = NP // 16
TE = 2048
DUMMY_ROW = N + 64

_mesh = plsc.VectorSubcoreMesh(
    core_axis_name="c", subcore_axis_name="s", num_cores=2, num_subcores=16)

_sc_params = pltpu.CompilerParams(use_tc_tiling_on_sc=False)


def _worker_id():
    return lax.axis_index("s") * 2 + lax.axis_index("c")


def _gather_body(h_hbm, src_hbm, hs_hbm, idx_v, rows0, rows1, sem0, sem1):
    wid = _worker_id()
    pltpu.sync_copy(src_hbm.at[wid], idx_v)
    ebase = wid * EPW

    def start(c, rows, sem):
        pltpu.async_copy(h_hbm.at[idx_v.at[c]], rows, sem)

    def wait(c, rows, sem):
        pltpu.make_async_copy(h_hbm.at[idx_v.at[c]], rows, sem).wait()

    start(0, rows0, sem0)

    def body(g, carry):
        c0 = 2 * g
        c1 = c0 + 1
        start(c1, rows1, sem1)
        wait(c0, rows0, sem0)
        pltpu.sync_copy(rows0, hs_hbm.at[pl.ds(ebase + c0 * CHUNK, CHUNK)])

        @pl.when(g < CPW // 2 - 1)
        def _():
            start(c0 + 2, rows0, sem0)

        wait(c1, rows1, sem1)
        pltpu.sync_copy(rows1, hs_hbm.at[pl.ds(ebase + c1 * CHUNK, CHUNK)])
        return carry

    lax.fori_loop(0, CPW // 2, body, 0)


_gather = pl.kernel(
    _gather_body,
    out_type=jax.ShapeDtypeStruct((EP, WIDTH), jnp.float32),
    mesh=_mesh,
    compiler_params=_sc_params,
    scratch_types=[
        pltpu.VMEM((CPW, CHUNK), jnp.int32),
        pltpu.VMEM((CHUNK, WIDTH), jnp.float32),
        pltpu.VMEM((CHUNK, WIDTH), jnp.float32),
        pltpu.SemaphoreType.DMA,
        pltpu.SemaphoreType.DMA,
    ],
)


def _zero_acc_slice(zbuf, acc):
    tid = lax.axis_index("s")

    def zf(r, carry):
        zbuf[r] = jnp.zeros((WIDTH,), jnp.float32)
        return carry

    lax.fori_loop(0, CHUNK, zf, 0)
    row0 = tid * ROWS_PER_TILE
    for j in range(ROWS_PER_TILE // CHUNK):
        pltpu.sync_copy(zbuf, acc.at[pl.ds(row0 + j * CHUNK, CHUNK)])


def _write_acc_out(out_hbm, bounce, acc):
    cid = lax.axis_index("c")
    tid = lax.axis_index("s")
    row0 = tid * ROWS_PER_TILE
    for j in range(ROWS_PER_TILE // CHUNK):
        r = row0 + j * CHUNK
        pltpu.sync_copy(acc.at[pl.ds(r, CHUNK)], bounce)
        pltpu.sync_copy(bounce, out_hbm.at[cid, pl.ds(r, CHUNK)])


def _scatter_body(msg_hbm, dst_hbm, out_hbm, idx_v, m0, m1, sem0, sem1, acc):
    wid = _worker_id()
    _zero_acc_slice(m0, acc)
    plsc.subcore_barrier()
    pltpu.sync_copy(dst_hbm.at[wid], idx_v)
    ebase = wid * EPW

    def start(c, rows, sem):
        pltpu.async_copy(msg_hbm.at[pl.ds(ebase + c * CHUNK, CHUNK)], rows, sem)

    def wait(c, rows, sem):
        pltpu.make_async_copy(
            msg_hbm.at[pl.ds(ebase + c * CHUNK, CHUNK)], rows, sem).wait()

    start(0, m0, sem0)

    def body(g, carry):
        c0 = 2 * g
        c1 = c0 + 1
        start(c1, m1, sem1)
        wait(c0, m0, sem0)
        pltpu.sync_copy(m0, acc.at[idx_v.at[c0]], add=True)

        @pl.when(g < CPW // 2 - 1)
        def _():
            start(c0 + 2, m0, sem0)

        wait(c1, m1, sem1)
        pltpu.sync_copy(m1, acc.at[idx_v.at[c1]], add=True)
        return carry

    lax.fori_loop(0, CPW // 2, body, 0)
    plsc.subcore_barrier()
    _write_acc_out(out_hbm, m0, acc)


_scatter = pl.kernel(
    _scatter_body,
    out_type=jax.ShapeDtypeStruct((2, NP, WIDTH), jnp.float32),
    mesh=_mesh,
    compiler_params=_sc_params,
    scratch_types=[
        pltpu.VMEM((CPW, CHUNK), jnp.int32),
        pltpu.VMEM((CHUNK, WIDTH), jnp.float32),
        pltpu.VMEM((CHUNK, WIDTH), jnp.float32),
        pltpu.SemaphoreType.DMA,
        pltpu.SemaphoreType.DMA,
        pltpu.VMEM_SHARED((NP, WIDTH), jnp.float32),
    ],
)


def _counts_body(dst_hbm, out_hbm, idx_v, ones_v, bounce, acc):
    wid = _worker_id()
    _zero_acc_slice(bounce, acc)

    def of(r, carry):
        ones_v[r] = jnp.ones((WIDTH,), jnp.float32)
        return carry

    lax.fori_loop(0, CHUNK, of, 0)
    plsc.subcore_barrier()
    pltpu.sync_copy(dst_hbm.at[wid], idx_v)

    def body(c, carry):
        pltpu.sync_copy(ones_v, acc.at[idx_v.at[c]], add=True)
        return carry

    lax.fori_loop(0, CPW, body, 0)
    plsc.subcore_barrier()
    _write_acc_out(out_hbm, bounce, acc)


_counts = pl.kernel(
    _counts_body,
    out_type=jax.ShapeDtypeStruct((2, NP, WIDTH), jnp.float32),
    mesh=_mesh,
    compiler_params=_sc_params,
    scratch_types=[
        pltpu.VMEM((CPW, CHUNK), jnp.int32),
        pltpu.VMEM((CHUNK, WIDTH), jnp.float32),
        pltpu.VMEM((CHUNK, WIDTH), jnp.float32),
        pltpu.VMEM_SHARED((NP, WIDTH), jnp.float32),
    ],
)


def _prep_body(x_ref, w_ref, b_ref, cnt_ref, h0_ref, invc_ref):
    h0_ref[...] = x_ref[...] * w_ref[...] + b_ref[...]
    c = cnt_ref[0] + cnt_ref[1]
    invc_ref[...] = 1.0 / jnp.maximum(c, 1.0)


def _edge_body(ea_ref, hs_ref, k1_ref, b1_ref, k2_ref, b2_ref, k3_ref,
               b3_ref, r_ref, out_ref):
    f32 = jnp.float32
    ka = jnp.dot(ea_ref[...], k1_ref[...], preferred_element_type=f32)
    ka = jnp.maximum(ka + b1_ref[...], 0.0)
    ka = jnp.dot(ka, k2_ref[...], preferred_element_type=f32)
    ka = jnp.maximum(ka + b2_ref[...], 0.0)
    w = jnp.dot(ka, k3_ref[...], preferred_element_type=f32) + b3_ref[...]
    hr = jnp.dot(hs_ref[...], r_ref[...], preferred_element_type=f32)
    p = w * hr
    p = p[:, :128] + p[:, 128:]
    p = p[:, :64] + p[:, 64:]
    p = p[:, :32] + p[:, 32:]
    out_ref[...] = p[:, :16] + p[:, 16:]


def _combine_body(p_ref, invc_ref, out_ref):
    h = (p_ref[0] + p_ref[1]) * invc_ref[...]
    out_ref[...] = jnp.maximum(h, 0.0)


def _post_body(p_ref, invc_ref, w2_ref, b2_ref, w3_ref, b3_ref, out_ref):
    f32 = jnp.float32
    h = (p_ref[0] + p_ref[1]) * invc_ref[...]
    h = jnp.dot(h, w2_ref[...], preferred_element_type=f32) + b2_ref[...]
    h = jnp.maximum(h, 0.0)
    out_ref[...] = jnp.dot(h, w3_ref[...], preferred_element_type=f32) + b3_ref[...]


def _full(shape):
    return pl.BlockSpec(shape, lambda i: (0,) * len(shape))


_prep = pl.pallas_call(
    _prep_body,
    out_shape=[
        jax.ShapeDtypeStruct((NP, WIDTH), jnp.float32),
        jax.ShapeDtypeStruct((NP, WIDTH), jnp.float32),
    ],
)

_edge = pl.pallas_call(
    _edge_body,
    grid=(EP // TE,),
    in_specs=[
        pl.BlockSpec((TE, 8), lambda i: (i, 0)),
        pl.BlockSpec((TE, WIDTH), lambda i: (i, 0)),
        _full((8, 64)),
        _full((1, 64)),
        _full((64, 128)),
        _full((1, 128)),
        _full((128, 256)),
        _full((1, 256)),
        _full((WIDTH, 256)),
    ],
    out_specs=pl.BlockSpec((TE, WIDTH), lambda i: (i, 0)),
    out_shape=jax.ShapeDtypeStruct((EP, WIDTH), jnp.float32),
)

_combine = pl.pallas_call(
    _combine_body,
    out_shape=jax.ShapeDtypeStruct((NP, WIDTH), jnp.float32),
)

_post = pl.pallas_call(
    _post_body,
    out_shape=jax.ShapeDtypeStruct((NP, 1), jnp.float32),
)


def kernel(x, edge_index, edge_attr, fc1_w, fc1_b, k1_w, k1_b, k2_w, k2_b,
           k3_w, k3_b, fc2_w, fc2_b, fc3_w, fc3_b):
    f32 = jnp.float32
    src = edge_index[0]
    dst = edge_index[1]
    pad = EP - E
    src_p = jnp.concatenate([src, jnp.zeros((pad,), jnp.int32)]).reshape(NW, CPW, CHUNK)
    dst_p = jnp.concatenate(
        [dst, jnp.full((pad,), DUMMY_ROW, jnp.int32)]).reshape(NW, CPW, CHUNK)
    ea_p = jnp.pad(edge_attr, ((0, pad), (0, 4)))
    x_p = jnp.pad(x, ((0, NP - N), (0, 0)))
    k1p = jnp.pad(k1_w, ((0, 4), (0, 0)))

    r_mat = (jnp.arange(256)[None, :] // WIDTH ==
             jnp.arange(WIDTH)[:, None]).astype(f32)

    b1 = k1_b.reshape(1, -1)
    b2 = k2_b.reshape(1, -1)
    b3 = k3_b.reshape(1, -1)
    fb2 = fc2_b.reshape(1, -1)
    fb3 = fc3_b.reshape(1, -1)

    cnt = _counts(dst_p)
    h, invc = _prep(x_p, fc1_w.reshape(1, WIDTH), fc1_b.reshape(1, WIDTH), cnt)

    for k in range(DEPTH):
        hs = _gather(h, src_p)
        msg = _edge(ea_p, hs, k1p, b1, k2_w, b2, k3_w, b3, r_mat)
        parts = _scatter(msg, dst_p)
        if k != DEPTH - 1:
            h = _combine(parts, invc)

    out = _post(parts, invc, fc2_w, fb2, fc3_w, fb3)
    return out[:N]


# speedup vs baseline: 3.1351x; 1.0310x over previous
import functools

import jax
import jax.numpy as jnp
from jax import lax
from jax.experimental import pallas as pl
from jax.experimental.pallas import tpu as pltpu
from jax.experimental.pallas import tpu_sc as plsc

N = 10000
E = 320000
WIDTH = 16
KER_WIDTH = 128
DEPTH = 4

NW = 32
CHUNK = 128
CPW = 80
EPW = CHUNK * CPW
EP = NW * EPW
NP = 10240
ROWS_PER_TILE = NP // 16
TE = 2048
NH = 2
EH = EP // NH
CPW_H = CPW // NH
DUMMY_ROW = N + 64

_mesh = plsc.VectorSubcoreMesh(
    core_axis_name="c", subcore_axis_name="s", num_cores=2, num_subcores=16)

_sc_params = pltpu.CompilerParams(use_tc_tiling_on_sc=False)


def _worker_id():
    return lax.axis_index("s") * 2 + lax.axis_index("c")


def _gather_body(cpw, h_hbm, src_hbm, hs_hbm, idx_v, rows0, rows1, sem0, sem1):
    wid = _worker_id()
    pltpu.sync_copy(src_hbm.at[wid], idx_v)
    ebase = wid * cpw * CHUNK

    def start(c, rows, sem):
        pltpu.async_copy(h_hbm.at[idx_v.at[c]], rows, sem)

    def wait(c, rows, sem):
        pltpu.make_async_copy(h_hbm.at[idx_v.at[c]], rows, sem).wait()

    start(0, rows0, sem0)

    def body(g, carry):
        c0 = 2 * g
        c1 = c0 + 1
        start(c1, rows1, sem1)
        wait(c0, rows0, sem0)
        pltpu.sync_copy(rows0, hs_hbm.at[pl.ds(ebase + c0 * CHUNK, CHUNK)])

        @pl.when(g < cpw // 2 - 1)
        def _():
            start(c0 + 2, rows0, sem0)

        wait(c1, rows1, sem1)
        pltpu.sync_copy(rows1, hs_hbm.at[pl.ds(ebase + c1 * CHUNK, CHUNK)])
        return carry

    lax.fori_loop(0, cpw // 2, body, 0)


def _make_gather(cpw):
    return pl.kernel(
        functools.partial(_gather_body, cpw),
        out_type=jax.ShapeDtypeStruct((NW * cpw * CHUNK, WIDTH), jnp.float32),
        mesh=_mesh,
        compiler_params=_sc_params,
        scratch_types=[
            pltpu.VMEM((cpw, CHUNK), jnp.int32),
            pltpu.VMEM((CHUNK, WIDTH), jnp.float32),
            pltpu.VMEM((CHUNK, WIDTH), jnp.float32),
            pltpu.SemaphoreType.DMA,
            pltpu.SemaphoreType.DMA,
        ],
    )


def _zero_acc_slice(zbuf, acc):
    tid = lax.axis_index("s")

    def zf(r, carry):
        zbuf[r] = jnp.zeros((WIDTH,), jnp.float32)
        return carry

    lax.fori_loop(0, CHUNK, zf, 0)
    row0 = tid * ROWS_PER_TILE
    for j in range(ROWS_PER_TILE // CHUNK):
        pltpu.sync_copy(zbuf, acc.at[pl.ds(row0 + j * CHUNK, CHUNK)])


def _write_acc_out(out_hbm, bounce, acc):
    cid = lax.axis_index("c")
    tid = lax.axis_index("s")
    row0 = tid * ROWS_PER_TILE
    for j in range(ROWS_PER_TILE // CHUNK):
        r = row0 + j * CHUNK
        pltpu.sync_copy(acc.at[pl.ds(r, CHUNK)], bounce)
        pltpu.sync_copy(bounce, out_hbm.at[cid, pl.ds(r, CHUNK)])


def _scatter_body(cpw, msg_hbm, dst_hbm, out_hbm, idx_v, m0, m1, sem0, sem1, acc):
    wid = _worker_id()
    _zero_acc_slice(m0, acc)
    plsc.subcore_barrier()
    pltpu.sync_copy(dst_hbm.at[wid], idx_v)
    ebase = wid * cpw * CHUNK

    def start(c, rows, sem):
        pltpu.async_copy(msg_hbm.at[pl.ds(ebase + c * CHUNK, CHUNK)], rows, sem)

    def wait(c, rows, sem):
        pltpu.make_async_copy(
            msg_hbm.at[pl.ds(ebase + c * CHUNK, CHUNK)], rows, sem).wait()

    start(0, m0, sem0)

    def body(g, carry):
        c0 = 2 * g
        c1 = c0 + 1
        start(c1, m1, sem1)
        wait(c0, m0, sem0)
        pltpu.sync_copy(m0, acc.at[idx_v.at[c0]], add=True)

        @pl.when(g < cpw // 2 - 1)
        def _():
            start(c0 + 2, m0, sem0)

        wait(c1, m1, sem1)
        pltpu.sync_copy(m1, acc.at[idx_v.at[c1]], add=True)
        return carry

    lax.fori_loop(0, cpw // 2, body, 0)
    plsc.subcore_barrier()
    _write_acc_out(out_hbm, m0, acc)


def _make_scatter(cpw):
    return pl.kernel(
        functools.partial(_scatter_body, cpw),
        out_type=jax.ShapeDtypeStruct((2, NP, WIDTH), jnp.float32),
        mesh=_mesh,
        compiler_params=_sc_params,
        scratch_types=[
            pltpu.VMEM((cpw, CHUNK), jnp.int32),
            pltpu.VMEM((CHUNK, WIDTH), jnp.float32),
            pltpu.VMEM((CHUNK, WIDTH), jnp.float32),
            pltpu.SemaphoreType.DMA,
            pltpu.SemaphoreType.DMA,
            pltpu.VMEM_SHARED((NP, WIDTH), jnp.float32),
        ],
    )


def _counts_body(dst_hbm, out_hbm, idx_v, ones_v, bounce, acc):
    wid = _worker_id()
    _zero_acc_slice(bounce, acc)

    def of(r, carry):
        ones_v[r] = jnp.ones((WIDTH,), jnp.float32)
        return carry

    lax.fori_loop(0, CHUNK, of, 0)
    plsc.subcore_barrier()
    pltpu.sync_copy(dst_hbm.at[wid], idx_v)

    def body(c, carry):
        pltpu.sync_copy(ones_v, acc.at[idx_v.at[c]], add=True)
        return carry

    lax.fori_loop(0, CPW, body, 0)
    plsc.subcore_barrier()
    _write_acc_out(out_hbm, bounce, acc)


_counts = pl.kernel(
    _counts_body,
    out_type=jax.ShapeDtypeStruct((2, NP, WIDTH), jnp.float32),
    mesh=_mesh,
    compiler_params=_sc_params,
    scratch_types=[
        pltpu.VMEM((CPW, CHUNK), jnp.int32),
        pltpu.VMEM((CHUNK, WIDTH), jnp.float32),
        pltpu.VMEM((CHUNK, WIDTH), jnp.float32),
        pltpu.VMEM_SHARED((NP, WIDTH), jnp.float32),
    ],
)


def _prep_body(x_ref, w_ref, b_ref, cnt_ref, h0_ref, invc_ref):
    h0_ref[...] = x_ref[...] * w_ref[...] + b_ref[...]
    c = cnt_ref[0] + cnt_ref[1]
    invc_ref[...] = 1.0 / jnp.maximum(c, 1.0)


def _edge_body(ea_ref, hs_ref, k1_ref, b1_ref, k2_ref, b2_ref, k3_ref,
               b3_ref, r_ref, out_ref):
    f32 = jnp.float32
    ka = jnp.dot(ea_ref[...], k1_ref[...], preferred_element_type=f32)
    ka = jnp.maximum(ka + b1_ref[...], 0.0)
    ka = jnp.dot(ka, k2_ref[...], preferred_element_type=f32)
    ka = jnp.maximum(ka + b2_ref[...], 0.0)
    w = jnp.dot(ka, k3_ref[...], preferred_element_type=f32) + b3_ref[...]
    hr = jnp.dot(hs_ref[...], r_ref[...], preferred_element_type=f32)
    p = w * hr
    p = p[:, :128] + p[:, 128:]
    p = p[:, :64] + p[:, 64:]
    p = p[:, :32] + p[:, 32:]
    out_ref[...] = p[:, :16] + p[:, 16:]


def _combine_body(p_ref, q_ref, invc_ref, out_ref):
    h = (p_ref[0] + p_ref[1] + q_ref[0] + q_ref[1]) * invc_ref[...]
    out_ref[...] = jnp.maximum(h, 0.0)


def _post_body(p_ref, q_ref, invc_ref, w2_ref, b2_ref, w3_ref, b3_ref, out_ref):
    f32 = jnp.float32
    h = (p_ref[0] + p_ref[1] + q_ref[0] + q_ref[1]) * invc_ref[...]
    h = jnp.dot(h, w2_ref[...], preferred_element_type=f32) + b2_ref[...]
    h = jnp.maximum(h, 0.0)
    out_ref[...] = jnp.dot(h, w3_ref[...], preferred_element_type=f32) + b3_ref[...]


def _full(shape):
    return pl.BlockSpec(shape, lambda i: (0,) * len(shape))


_prep = pl.pallas_call(
    _prep_body,
    out_shape=[
        jax.ShapeDtypeStruct((NP, WIDTH), jnp.float32),
        jax.ShapeDtypeStruct((NP, WIDTH), jnp.float32),
    ],
)

_edge = pl.pallas_call(
    _edge_body,
    grid=(EH // TE,),
    in_specs=[
        pl.BlockSpec((TE, 8), lambda i: (i, 0)),
        pl.BlockSpec((TE, WIDTH), lambda i: (i, 0)),
        _full((8, 64)),
        _full((1, 64)),
        _full((64, 128)),
        _full((1, 128)),
        _full((128, 256)),
        _full((1, 256)),
        _full((WIDTH, 256)),
    ],
    out_specs=pl.BlockSpec((TE, WIDTH), lambda i: (i, 0)),
    out_shape=jax.ShapeDtypeStruct((EH, WIDTH), jnp.float32),
)

_gather_h = _make_gather(CPW_H)
_scatter_h = _make_scatter(CPW_H)

_combine = pl.pallas_call(
    _combine_body,
    out_shape=jax.ShapeDtypeStruct((NP, WIDTH), jnp.float32),
)

_post = pl.pallas_call(
    _post_body,
    out_shape=jax.ShapeDtypeStruct((NP, 1), jnp.float32),
)


def kernel(x, edge_index, edge_attr, fc1_w, fc1_b, k1_w, k1_b, k2_w, k2_b,
           k3_w, k3_b, fc2_w, fc2_b, fc3_w, fc3_b):
    f32 = jnp.float32
    src = edge_index[0]
    dst = edge_index[1]
    pad = EP - E
    src_p = jnp.concatenate([src, jnp.zeros((pad,), jnp.int32)])
    dst_p = jnp.concatenate([dst, jnp.full((pad,), DUMMY_ROW, jnp.int32)])
    src_h = [src_p[i * EH:(i + 1) * EH].reshape(NW, CPW_H, CHUNK) for i in range(NH)]
    dst_h = [dst_p[i * EH:(i + 1) * EH].reshape(NW, CPW_H, CHUNK) for i in range(NH)]
    dst_full = dst_p.reshape(NW, CPW, CHUNK)
    ea_p = jnp.pad(edge_attr, ((0, pad), (0, 4)))
    ea_h = [ea_p[i * EH:(i + 1) * EH] for i in range(NH)]
    x_p = jnp.pad(x, ((0, NP - N), (0, 0)))
    k1p = jnp.pad(k1_w, ((0, 4), (0, 0)))

    r_mat = (jnp.arange(256)[None, :] // WIDTH ==
             jnp.arange(WIDTH)[:, None]).astype(f32)

    b1 = k1_b.reshape(1, -1)
    b2 = k2_b.reshape(1, -1)
    b3 = k3_b.reshape(1, -1)
    fb2 = fc2_b.reshape(1, -1)
    fb3 = fc3_b.reshape(1, -1)

    cnt = _counts(dst_full)
    h, invc = _prep(x_p, fc1_w.reshape(1, WIDTH), fc1_b.reshape(1, WIDTH), cnt)

    for k in range(DEPTH):
        parts = []
        msgs = [None] * NH
        for i in range(NH):
            hs = _gather_h(h, src_h[i])
            msgs[i] = _edge(ea_h[i], hs, k1p, b1, k2_w, b2, k3_w, b3, r_mat)
        for i in range(NH):
            parts.append(_scatter_h(msgs[i], dst_h[i]))
        if k != DEPTH - 1:
            h = _combine(parts[0], parts[1], invc)

    out = _post(parts[0], parts[1], invc, fc2_w, fb2, fc3_w, fb3)
    return out[:N]
